# baseline (speedup 1.0000x reference)
import functools

import jax
import jax.numpy as jnp
from jax import lax
from jax.experimental import pallas as pl
from jax.experimental.pallas import tpu as pltpu
from jax.experimental.pallas import tpu_sc as plsc

NPTS = 131072
NLEV = 16
TSIZE = 524288
FDIM = 2
ENC_DIM = NLEV * FDIM
MIN_RES = 16.0
MAX_RES = 1024.0

NW = 32
PTS_PER_W = NPTS // NW
GRP = 16
CHUNK = 64
GPC = CHUNK // GRP
NCHUNK = PTS_PER_W // CHUNK
ROWS = CHUNK * NLEV * 8 * FDIM
FOFF = NLEV * TSIZE

P1 = 2654435761
P2 = 805459861
MASK = TSIZE - 1


def _enc_sc(posx, posy, posz, scal_b, table_flat):
    mesh = plsc.VectorSubcoreMesh(core_axis_name="c", subcore_axis_name="s")

    @functools.partial(
        pl.kernel,
        out_type=jax.ShapeDtypeStruct((ENC_DIM, NPTS), jnp.float32),
        mesh=mesh,
        scratch_types=[
            pltpu.VMEM((CHUNK,), jnp.float32),
            pltpu.VMEM((CHUNK,), jnp.float32),
            pltpu.VMEM((CHUNK,), jnp.float32),
            pltpu.VMEM((NLEV * 16,), jnp.float32),
            pltpu.VMEM((ROWS,), jnp.int32),
            pltpu.VMEM((ROWS,), jnp.int32),
            pltpu.VMEM((ROWS,), jnp.float32),
            pltpu.VMEM((ROWS,), jnp.float32),
            pltpu.VMEM((CHUNK * 48,), jnp.float32),
            pltpu.VMEM((CHUNK * 48,), jnp.float32),
            pltpu.VMEM((ENC_DIM, 2 * CHUNK), jnp.float32),
            pltpu.SemaphoreType.DMA,
            pltpu.SemaphoreType.DMA,
        ],
    )
    def body(posx_h, posy_h, posz_h, scal_h, table_h, out_h,
             px, py, pz, scal_v, idx_a, idx_b, rows_a, rows_b,
             frac_a, frac_b, enc_v, sem_a, sem_b):
        wid = lax.axis_index("s") * 2 + lax.axis_index("c")
        base_w = wid * PTS_PER_W
        pltpu.sync_copy(scal_h, scal_v)
        iota = lax.iota(jnp.int32, 16)
        zcol = iota * 0
        ocol = zcol + 1

        def phase_a(ci, idx_v, frac_v):
            base = base_w + ci * CHUNK
            pltpu.sync_copy(posx_h.at[pl.ds(base, CHUNK)], px)
            pltpu.sync_copy(posy_h.at[pl.ds(base, CHUNK)], py)
            pltpu.sync_copy(posz_h.at[pl.ds(base, CHUNK)], pz)

            def grp_body(g, c2):
                xs = px[pl.ds(g * GRP, 16)]
                ys = py[pl.ds(g * GRP, 16)]
                zs = pz[pl.ds(g * GRP, 16)]
                for l in range(NLEV):
                    scal = scal_v[pl.ds(l * 16, 16)]
                    sx = xs * scal
                    sy = ys * scal
                    sz = zs * scal
                    fxi = sx.astype(jnp.int32)
                    fyi = sy.astype(jnp.int32)
                    fzi = sz.astype(jnp.int32)
                    fxf = fxi.astype(jnp.float32)
                    fyf = fyi.astype(jnp.float32)
                    fzf = fzi.astype(jnp.float32)
                    cxi = fxi + jnp.where(sx > fxf, ocol, zcol)
                    cyi = fyi + jnp.where(sy > fyf, ocol, zcol)
                    czi = fzi + jnp.where(sz > fzf, ocol, zcol)
                    fb = (g * 48 + l * 3) * 16
                    frac_v[pl.ds(fb, 16)] = sx - fxf
                    frac_v[pl.ds(fb + 16, 16)] = sy - fyf
                    frac_v[pl.ds(fb + 32, 16)] = sz - fzf
                    axc = cxi.astype(jnp.uint32)
                    axf = fxi.astype(jnp.uint32)
                    ayc = cyi.astype(jnp.uint32) * jnp.uint32(P1)
                    ayf = fyi.astype(jnp.uint32) * jnp.uint32(P1)
                    azc = czi.astype(jnp.uint32) * jnp.uint32(P2)
                    azf = fzi.astype(jnp.uint32) * jnp.uint32(P2)
                    off = jnp.uint32(l * TSIZE)
                    msk = jnp.uint32(MASK)
                    rb = (g * NLEV + l) * 8 * 32
                    hs = (
                        axc ^ ayc ^ azc, axc ^ ayc ^ azf,
                        axc ^ ayf ^ azc, axf ^ ayc ^ azc,
                        axc ^ ayf ^ azf, axf ^ ayc ^ azf,
                        axf ^ ayf ^ azc, axf ^ ayf ^ azf,
                    )
                    for c in range(8):
                        h0 = ((hs[c] & msk) + off).astype(jnp.int32)
                        idx_v[pl.ds(rb + c * 32, 16)] = h0
                        idx_v[pl.ds(rb + c * 32 + 16, 16)] = h0 + FOFF
                return c2

            lax.fori_loop(0, GPC, grp_body, 0)

        HALF = ROWS // 2

        def fire(idx_v, rows_v, sem):
            pltpu.async_copy(table_h.at[idx_v.at[pl.ds(0, HALF)]],
                             rows_v.at[pl.ds(0, HALF)], sem)
            pltpu.async_copy(table_h.at[idx_v.at[pl.ds(HALF, HALF)]],
                             rows_v.at[pl.ds(HALF, HALF)], sem)

        def drain(idx_v, rows_v, sem):
            pltpu.make_async_copy(table_h.at[idx_v.at[pl.ds(0, HALF)]],
                                  rows_v.at[pl.ds(0, HALF)], sem).wait()
            pltpu.make_async_copy(table_h.at[idx_v.at[pl.ds(HALF, HALF)]],
                                  rows_v.at[pl.ds(HALF, HALF)], sem).wait()

        def phase_b(ci, rows_v, frac_v, col, flush):

            def grp_body(g, c2):
                for l in range(NLEV):
                    rb = (g * NLEV + l) * 8 * 32
                    f = []
                    for c in range(8):
                        f.append((rows_v[pl.ds(rb + c * 32, 16)],
                                  rows_v[pl.ds(rb + c * 32 + 16, 16)]))
                    wb = g * 48 * 16
                    w0 = frac_v[pl.ds(wb + l * 16, 16)]
                    w1 = frac_v[pl.ds(wb + (16 + l) * 16, 16)]
                    w2 = frac_v[pl.ds(wb + (32 + l) * 16, 16)]

                    def lerp(a, b, w):
                        return b + (a - b) * w

                    for ft in range(FDIM):
                        f03 = lerp(f[0][ft], f[3][ft], w0)
                        f12 = lerp(f[1][ft], f[2][ft], w0)
                        f56 = lerp(f[5][ft], f[6][ft], w0)
                        f47 = lerp(f[4][ft], f[7][ft], w0)
                        f0312 = lerp(f03, f12, w1)
                        f4756 = lerp(f47, f56, w1)
                        e = lerp(f0312, f4756, w2)
                        enc_v[2 * l + ft, pl.ds(col + g * GRP, 16)] = e
                return c2

            lax.fori_loop(0, GPC, grp_body, 0)
            if flush:
                base2 = base_w + (ci - 1) * CHUNK
                pltpu.sync_copy(enc_v, out_h.at[:, pl.ds(base2, 2 * CHUNK)])

        phase_a(0, idx_a, frac_a)
        fire(idx_a, rows_a, sem_a)

        def pipe_body(i, carry):
            c1 = 2 * i + 1
            phase_a(c1, idx_b, frac_b)
            fire(idx_b, rows_b, sem_b)
            drain(idx_a, rows_a, sem_a)
            phase_b(c1 - 1, rows_a, frac_a, 0, False)
            phase_a(c1 + 1, idx_a, frac_a)
            fire(idx_a, rows_a, sem_a)
            drain(idx_b, rows_b, sem_b)
            phase_b(c1, rows_b, frac_b, CHUNK, True)
            return carry

        lax.fori_loop(0, (NCHUNK - 2) // 2, pipe_body, 0)
        phase_a(NCHUNK - 1, idx_b, frac_b)
        fire(idx_b, rows_b, sem_b)
        drain(idx_a, rows_a, sem_a)
        phase_b(NCHUNK - 2, rows_a, frac_a, 0, False)
        drain(idx_b, rows_b, sem_b)
        phase_b(NCHUNK - 1, rows_b, frac_b, CHUNK, True)

    return body(posx, posy, posz, scal_b, table_flat)


def _mlp_body(enc_ref, dir_ref, w1, b1, w2, b2, w3, b3, w4, b4, w5, b5,
              dens_ref, col_ref):
    enc_t = enc_ref[...]
    h = jnp.maximum(
        lax.dot_general(enc_t, w1[...], (((0,), (0,)), ((), ())),
                        preferred_element_type=jnp.float32) + b1[...], 0.0)
    dens = jnp.dot(h, w2[...], preferred_element_type=jnp.float32) + b2[...]
    dens_ref[...] = dens

    d = dir_ref[...]
    x = d[0:1, :]
    y = d[1:2, :]
    z = d[2:3, :]
    xx = x * x
    yy = y * y
    zz = z * z
    one = jnp.ones_like(x)
    sh_t = jnp.concatenate([
        0.28209479177387814 * one,
        0.4886025119029199 * y,
        0.4886025119029199 * z,
        0.4886025119029199 * x,
        1.0925484305920792 * x * y,
        1.0925484305920792 * y * z,
        0.9461746957575601 * zz - 0.31539156525252,
        1.0925484305920792 * x * z,
        0.5462742152960396 * (xx - yy),
        0.5900435899266435 * y * (3.0 * xx - yy),
        2.890611442640554 * x * y * z,
        0.4570457994644658 * y * (5.0 * zz - 1.0),
        0.3731763325901154 * z * (5.0 * zz - 3.0),
        0.4570457994644658 * x * (5.0 * zz - 1.0),
        1.445305721320277 * z * (xx - yy),
        0.5900435899266435 * x * (xx - 3.0 * yy),
    ], axis=0)

    w3m = w3[...]
    x1 = (jnp.dot(dens, w3m[0:16, :], preferred_element_type=jnp.float32)
          + lax.dot_general(sh_t, w3m[16:32, :], (((0,), (0,)), ((), ())),
                            preferred_element_type=jnp.float32)
          + b3[...])
    x1 = jnp.maximum(x1, 0.0)
    x2 = jnp.maximum(
        jnp.dot(x1, w4[...], preferred_element_type=jnp.float32) + b4[...], 0.0)
    x3 = jnp.dot(x2, w5[...], preferred_element_type=jnp.float32) + b5[...]
    col_ref[...] = jax.nn.sigmoid(x3)


def _mlp_tc(enc, dirt, W1, b1, W2, b2, W3, b3, W4, b4, W5p, b5p):
    BLK = 1024
    grid = (NPTS // BLK,)
    full = lambda shape: pl.BlockSpec(shape, lambda i: (0, 0))
    return pl.pallas_call(
        _mlp_body,
        grid=grid,
        in_specs=[
            pl.BlockSpec((ENC_DIM, BLK), lambda i: (0, i)),
            pl.BlockSpec((8, BLK), lambda i: (0, i)),
            full(W1.shape), full((1, 64)),
            full(W2.shape), full((1, 16)),
            full(W3.shape), full((1, 64)),
            full(W4.shape), full((1, 64)),
            full(W5p.shape), full((1, 8)),
        ],
        out_specs=[
            pl.BlockSpec((BLK, 16), lambda i: (i, 0)),
            pl.BlockSpec((BLK, 8), lambda i: (i, 0)),
        ],
        out_shape=[
            jax.ShapeDtypeStruct((NPTS, 16), jnp.float32),
            jax.ShapeDtypeStruct((NPTS, 8), jnp.float32),
        ],
    )(enc, dirt, W1, b1.reshape(1, 64), W2, b2.reshape(1, 16),
      W3, b3.reshape(1, 64), W4, b4.reshape(1, 64), W5p, b5p.reshape(1, 8))


def kernel(position, direction, hash_table, W1, b1, W2, b2, W3, b3, W4, b4,
           W5, b5):
    levels = jnp.arange(NLEV)
    growth = jnp.exp((jnp.log(MAX_RES) - jnp.log(MIN_RES)) / (NLEV - 1))
    scalings = jnp.floor(MIN_RES * growth ** levels).astype(jnp.float32)
    scal_b = jnp.repeat(scalings, 16)
    table_flat = hash_table.reshape(-1)
    posx = position[:, 0]
    posy = position[:, 1]
    posz = position[:, 2]

    enc_t = _enc_sc(posx, posy, posz, scal_b, table_flat)

    dirt = jnp.pad(direction.T, ((0, 5), (0, 0)))
    W5p = jnp.pad(W5, ((0, 0), (0, 5)))
    b5p = jnp.pad(b5, (0, 5))
    dens, colp = _mlp_tc(enc_t, dirt, W1, b1, W2, b2, W3, b3, W4, b4, W5p, b5p)
    return dens, colp[:, :3]


# speedup vs baseline: 1.5880x; 1.5880x over previous
import functools

import jax
import jax.numpy as jnp
from jax import lax
from jax.experimental import pallas as pl
from jax.experimental.pallas import tpu as pltpu
from jax.experimental.pallas import tpu_sc as plsc

NPTS = 131072
NLEV = 16
TSIZE = 524288
FDIM = 2
ENC_DIM = NLEV * FDIM
MIN_RES = 16.0
MAX_RES = 1024.0

NW = 32
PTS_PER_W = NPTS // NW
GRP = 16
CHUNK = 128
GPC = CHUNK // GRP
NCHUNK = PTS_PER_W // CHUNK
ROWS = CHUNK * NLEV * 8

P1 = 2654435761
P2 = 805459861
MASK = TSIZE - 1


def _enc_sc(posx, posy, posz, scal_b, table_packed):
    mesh = plsc.VectorSubcoreMesh(core_axis_name="c", subcore_axis_name="s")

    @functools.partial(
        pl.kernel,
        out_type=jax.ShapeDtypeStruct((ENC_DIM, NPTS), jnp.float32),
        mesh=mesh,
        scratch_types=[
            pltpu.VMEM((CHUNK,), jnp.float32),
            pltpu.VMEM((CHUNK,), jnp.float32),
            pltpu.VMEM((CHUNK,), jnp.float32),
            pltpu.VMEM((NLEV * 16,), jnp.float32),
            pltpu.VMEM((ROWS,), jnp.int32),
            pltpu.VMEM((ROWS,), jnp.int32),
            pltpu.VMEM((ROWS,), jnp.int32),
            pltpu.VMEM((ROWS,), jnp.int32),
            pltpu.VMEM((CHUNK * 48,), jnp.float32),
            pltpu.VMEM((CHUNK * 48,), jnp.float32),
            pltpu.VMEM((ENC_DIM, CHUNK), jnp.float32),
            pltpu.SemaphoreType.DMA,
            pltpu.SemaphoreType.DMA,
        ],
    )
    def body(posx_h, posy_h, posz_h, scal_h, table_h, out_h,
             px, py, pz, scal_v, idx_a, idx_b, rows_a, rows_b,
             frac_a, frac_b, enc_v, sem_a, sem_b):
        wid = lax.axis_index("s") * 2 + lax.axis_index("c")
        base_w = wid * PTS_PER_W
        pltpu.sync_copy(scal_h, scal_v)
        iota = lax.iota(jnp.int32, 16)
        zcol = iota * 0
        ocol = zcol + 1

        def phase_a(ci, idx_v, frac_v):
            base = base_w + ci * CHUNK
            pltpu.sync_copy(posx_h.at[pl.ds(base, CHUNK)], px)
            pltpu.sync_copy(posy_h.at[pl.ds(base, CHUNK)], py)
            pltpu.sync_copy(posz_h.at[pl.ds(base, CHUNK)], pz)

            def grp_body(g, c2):
                xs = px[pl.ds(g * GRP, 16)]
                ys = py[pl.ds(g * GRP, 16)]
                zs = pz[pl.ds(g * GRP, 16)]
                for l in range(NLEV):
                    scal = scal_v[pl.ds(l * 16, 16)]
                    sx = xs * scal
                    sy = ys * scal
                    sz = zs * scal
                    fxi = sx.astype(jnp.int32)
                    fyi = sy.astype(jnp.int32)
                    fzi = sz.astype(jnp.int32)
                    fxf = fxi.astype(jnp.float32)
                    fyf = fyi.astype(jnp.float32)
                    fzf = fzi.astype(jnp.float32)
                    cxi = fxi + jnp.where(sx > fxf, ocol, zcol)
                    cyi = fyi + jnp.where(sy > fyf, ocol, zcol)
                    czi = fzi + jnp.where(sz > fzf, ocol, zcol)
                    fb = (g * 48 + l * 3) * 16
                    frac_v[pl.ds(fb, 16)] = sx - fxf
                    frac_v[pl.ds(fb + 16, 16)] = sy - fyf
                    frac_v[pl.ds(fb + 32, 16)] = sz - fzf
                    axc = cxi.astype(jnp.uint32)
                    axf = fxi.astype(jnp.uint32)
                    ayc = cyi.astype(jnp.uint32) * jnp.uint32(P1)
                    ayf = fyi.astype(jnp.uint32) * jnp.uint32(P1)
                    azc = czi.astype(jnp.uint32) * jnp.uint32(P2)
                    azf = fzi.astype(jnp.uint32) * jnp.uint32(P2)
                    off = jnp.uint32(l * TSIZE)
                    msk = jnp.uint32(MASK)
                    rb = (g * NLEV + l) * 8 * 16
                    hs = (
                        axc ^ ayc ^ azc, axc ^ ayc ^ azf,
                        axc ^ ayf ^ azc, axf ^ ayc ^ azc,
                        axc ^ ayf ^ azf, axf ^ ayc ^ azf,
                        axf ^ ayf ^ azc, axf ^ ayf ^ azf,
                    )
                    for c in range(8):
                        h0 = ((hs[c] & msk) + off).astype(jnp.int32)
                        idx_v[pl.ds(rb + c * 16, 16)] = h0
                return c2

            lax.fori_loop(0, GPC, grp_body, 0)

        def fire(idx_v, rows_v, sem):
            pltpu.async_copy(table_h.at[idx_v], rows_v, sem)

        def drain(idx_v, rows_v, sem):
            pltpu.make_async_copy(table_h.at[idx_v], rows_v, sem).wait()

        hi_msk = jnp.uint32(0xFFFF0000)

        def phase_b(ci, rows_v, frac_v):
            base = base_w + ci * CHUNK

            def grp_body(g, c2):
                for l in range(NLEV):
                    rb = (g * NLEV + l) * 8 * 16
                    f = []
                    for c in range(8):
                        w = lax.bitcast_convert_type(
                            rows_v[pl.ds(rb + c * 16, 16)], jnp.uint32)
                        f.append((
                            lax.bitcast_convert_type(w & hi_msk, jnp.float32),
                            lax.bitcast_convert_type(w << 16, jnp.float32)))
                    wb = g * 48 * 16
                    w0 = frac_v[pl.ds(wb + l * 16, 16)]
                    w1 = frac_v[pl.ds(wb + (16 + l) * 16, 16)]
                    w2 = frac_v[pl.ds(wb + (32 + l) * 16, 16)]

                    def lerp(a, b, w):
                        return b + (a - b) * w

                    for ft in range(FDIM):
                        f03 = lerp(f[0][ft], f[3][ft], w0)
                        f12 = lerp(f[1][ft], f[2][ft], w0)
                        f56 = lerp(f[5][ft], f[6][ft], w0)
                        f47 = lerp(f[4][ft], f[7][ft], w0)
                        f0312 = lerp(f03, f12, w1)
                        f4756 = lerp(f47, f56, w1)
                        e = lerp(f0312, f4756, w2)
                        enc_v[2 * l + ft, pl.ds(g * GRP, 16)] = e
                return c2

            lax.fori_loop(0, GPC, grp_body, 0)
            pltpu.sync_copy(enc_v, out_h.at[:, pl.ds(base, CHUNK)])

        phase_a(0, idx_a, frac_a)
        fire(idx_a, rows_a, sem_a)

        def pipe_body(i, carry):
            c1 = 2 * i + 1
            phase_a(c1, idx_b, frac_b)
            fire(idx_b, rows_b, sem_b)
            drain(idx_a, rows_a, sem_a)
            phase_b(c1 - 1, rows_a, frac_a)
            phase_a(c1 + 1, idx_a, frac_a)
            fire(idx_a, rows_a, sem_a)
            drain(idx_b, rows_b, sem_b)
            phase_b(c1, rows_b, frac_b)
            return carry

        lax.fori_loop(0, (NCHUNK - 2) // 2, pipe_body, 0)
        phase_a(NCHUNK - 1, idx_b, frac_b)
        fire(idx_b, rows_b, sem_b)
        drain(idx_a, rows_a, sem_a)
        phase_b(NCHUNK - 2, rows_a, frac_a)
        drain(idx_b, rows_b, sem_b)
        phase_b(NCHUNK - 1, rows_b, frac_b)

    return body(posx, posy, posz, scal_b, table_packed)


def _mlp_body(enc_ref, dir_ref, w1, b1, w2, b2, w3, b3, w4, b4, w5, b5,
              dens_ref, col_ref):
    enc_t = enc_ref[...]
    h = jnp.maximum(
        lax.dot_general(enc_t, w1[...], (((0,), (0,)), ((), ())),
                        preferred_element_type=jnp.float32) + b1[...], 0.0)
    dens = jnp.dot(h, w2[...], preferred_element_type=jnp.float32) + b2[...]
    dens_ref[...] = dens

    d = dir_ref[...]
    x = d[0:1, :]
    y = d[1:2, :]
    z = d[2:3, :]
    xx = x * x
    yy = y * y
    zz = z * z
    one = jnp.ones_like(x)
    sh_t = jnp.concatenate([
        0.28209479177387814 * one,
        0.4886025119029199 * y,
        0.4886025119029199 * z,
        0.4886025119029199 * x,
        1.0925484305920792 * x * y,
        1.0925484305920792 * y * z,
        0.9461746957575601 * zz - 0.31539156525252,
        1.0925484305920792 * x * z,
        0.5462742152960396 * (xx - yy),
        0.5900435899266435 * y * (3.0 * xx - yy),
        2.890611442640554 * x * y * z,
        0.4570457994644658 * y * (5.0 * zz - 1.0),
        0.3731763325901154 * z * (5.0 * zz - 3.0),
        0.4570457994644658 * x * (5.0 * zz - 1.0),
        1.445305721320277 * z * (xx - yy),
        0.5900435899266435 * x * (xx - 3.0 * yy),
    ], axis=0)

    w3m = w3[...]
    x1 = (jnp.dot(dens, w3m[0:16, :], preferred_element_type=jnp.float32)
          + lax.dot_general(sh_t, w3m[16:32, :], (((0,), (0,)), ((), ())),
                            preferred_element_type=jnp.float32)
          + b3[...])
    x1 = jnp.maximum(x1, 0.0)
    x2 = jnp.maximum(
        jnp.dot(x1, w4[...], preferred_element_type=jnp.float32) + b4[...], 0.0)
    x3 = jnp.dot(x2, w5[...], preferred_element_type=jnp.float32) + b5[...]
    col_ref[...] = jax.nn.sigmoid(x3)


def _mlp_tc(enc, dirt, W1, b1, W2, b2, W3, b3, W4, b4, W5p, b5p):
    BLK = 1024
    grid = (NPTS // BLK,)
    full = lambda shape: pl.BlockSpec(shape, lambda i: (0, 0))
    return pl.pallas_call(
        _mlp_body,
        grid=grid,
        in_specs=[
            pl.BlockSpec((ENC_DIM, BLK), lambda i: (0, i)),
            pl.BlockSpec((8, BLK), lambda i: (0, i)),
            full(W1.shape), full((1, 64)),
            full(W2.shape), full((1, 16)),
            full(W3.shape), full((1, 64)),
            full(W4.shape), full((1, 64)),
            full(W5p.shape), full((1, 8)),
        ],
        out_specs=[
            pl.BlockSpec((BLK, 16), lambda i: (i, 0)),
            pl.BlockSpec((BLK, 8), lambda i: (i, 0)),
        ],
        out_shape=[
            jax.ShapeDtypeStruct((NPTS, 16), jnp.float32),
            jax.ShapeDtypeStruct((NPTS, 8), jnp.float32),
        ],
    )(enc, dirt, W1, b1.reshape(1, 64), W2, b2.reshape(1, 16),
      W3, b3.reshape(1, 64), W4, b4.reshape(1, 64), W5p, b5p.reshape(1, 8))


def kernel(position, direction, hash_table, W1, b1, W2, b2, W3, b3, W4, b4,
           W5, b5):
    levels = jnp.arange(NLEV)
    growth = jnp.exp((jnp.log(MAX_RES) - jnp.log(MIN_RES)) / (NLEV - 1))
    scalings = jnp.floor(MIN_RES * growth ** levels).astype(jnp.float32)
    scal_b = jnp.repeat(scalings, 16)
    u0 = lax.bitcast_convert_type(
        hash_table[0].astype(jnp.bfloat16), jnp.uint16).astype(jnp.uint32)
    u1 = lax.bitcast_convert_type(
        hash_table[1].astype(jnp.bfloat16), jnp.uint16).astype(jnp.uint32)
    table_packed = lax.bitcast_convert_type((u0 << 16) | u1, jnp.int32)
    posx = position[:, 0]
    posy = position[:, 1]
    posz = position[:, 2]

    enc_t = _enc_sc(posx, posy, posz, scal_b, table_packed)

    dirt = jnp.pad(direction.T, ((0, 5), (0, 0)))
    W5p = jnp.pad(W5, ((0, 0), (0, 5)))
    b5p = jnp.pad(b5, (0, 5))
    dens, colp = _mlp_tc(enc_t, dirt, W1, b1, W2, b2, W3, b3, W4, b4, W5p, b5p)
    return dens, colp[:, :3]


# speedup vs baseline: 1.6444x; 1.0355x over previous
import functools

import jax
import jax.numpy as jnp
from jax import lax
from jax.experimental import pallas as pl
from jax.experimental.pallas import tpu as pltpu
from jax.experimental.pallas import tpu_sc as plsc

NPTS = 131072
NLEV = 16
TSIZE = 524288
FDIM = 2
ENC_DIM = NLEV * FDIM
MIN_RES = 16.0
MAX_RES = 1024.0

NW = 32
PTS_PER_W = NPTS // NW
GRP = 16
CHUNK = 128
GPC = CHUNK // GRP
NCHUNK = PTS_PER_W // CHUNK
ROWS = CHUNK * NLEV * 8

P1 = 2654435761
P2 = 805459861
MASK = TSIZE - 1


def _enc_sc(posx, posy, posz, scal_b, table_packed):
    mesh = plsc.VectorSubcoreMesh(core_axis_name="c", subcore_axis_name="s")

    @functools.partial(
        pl.kernel,
        out_type=jax.ShapeDtypeStruct((ENC_DIM, NPTS), jnp.float32),
        mesh=mesh,
        scratch_types=[
            pltpu.VMEM((CHUNK,), jnp.float32),
            pltpu.VMEM((CHUNK,), jnp.float32),
            pltpu.VMEM((CHUNK,), jnp.float32),
            pltpu.VMEM((NLEV * 16,), jnp.float32),
            pltpu.VMEM((ROWS,), jnp.int32),
            pltpu.VMEM((ROWS,), jnp.int32),
            pltpu.VMEM((ROWS,), jnp.int32),
            pltpu.VMEM((ROWS,), jnp.int32),
            pltpu.VMEM((CHUNK * 48,), jnp.float32),
            pltpu.VMEM((CHUNK * 48,), jnp.float32),
            pltpu.VMEM((ENC_DIM, CHUNK), jnp.float32),
            pltpu.SemaphoreType.DMA,
            pltpu.SemaphoreType.DMA,
        ],
    )
    def body(posx_h, posy_h, posz_h, scal_h, table_h, out_h,
             px, py, pz, scal_v, idx_a, idx_b, rows_a, rows_b,
             frac_a, frac_b, enc_v, sem_a, sem_b):
        wid = lax.axis_index("s") * 2 + lax.axis_index("c")
        base_w = wid * PTS_PER_W
        pltpu.sync_copy(scal_h, scal_v)
        iota = lax.iota(jnp.int32, 16)
        zcol = iota * 0
        ocol = zcol + 1

        def phase_a(ci, idx_v, frac_v):
            base = base_w + ci * CHUNK
            pltpu.sync_copy(posx_h.at[pl.ds(base, CHUNK)], px)
            pltpu.sync_copy(posy_h.at[pl.ds(base, CHUNK)], py)
            pltpu.sync_copy(posz_h.at[pl.ds(base, CHUNK)], pz)

            def grp_body(g, c2):
                xs = px[pl.ds(g * GRP, 16)]
                ys = py[pl.ds(g * GRP, 16)]
                zs = pz[pl.ds(g * GRP, 16)]
                for l in range(NLEV):
                    scal = scal_v[pl.ds(l * 16, 16)]
                    sx = xs * scal
                    sy = ys * scal
                    sz = zs * scal
                    fxi = sx.astype(jnp.int32)
                    fyi = sy.astype(jnp.int32)
                    fzi = sz.astype(jnp.int32)
                    fxf = fxi.astype(jnp.float32)
                    fyf = fyi.astype(jnp.float32)
                    fzf = fzi.astype(jnp.float32)
                    cxi = fxi + jnp.where(sx > fxf, ocol, zcol)
                    cyi = fyi + jnp.where(sy > fyf, ocol, zcol)
                    czi = fzi + jnp.where(sz > fzf, ocol, zcol)
                    fb = (g * 48 + l * 3) * 16
                    frac_v[pl.ds(fb, 16)] = sx - fxf
                    frac_v[pl.ds(fb + 16, 16)] = sy - fyf
                    frac_v[pl.ds(fb + 32, 16)] = sz - fzf
                    axc = cxi.astype(jnp.uint32)
                    axf = fxi.astype(jnp.uint32)
                    ayc = cyi.astype(jnp.uint32) * jnp.uint32(P1)
                    ayf = fyi.astype(jnp.uint32) * jnp.uint32(P1)
                    azc = czi.astype(jnp.uint32) * jnp.uint32(P2)
                    azf = fzi.astype(jnp.uint32) * jnp.uint32(P2)
                    off = jnp.uint32(l * TSIZE)
                    msk = jnp.uint32(MASK)
                    rb = (g * NLEV + l) * 8 * 16
                    hs = (
                        axc ^ ayc ^ azc, axc ^ ayc ^ azf,
                        axc ^ ayf ^ azc, axf ^ ayc ^ azc,
                        axc ^ ayf ^ azf, axf ^ ayc ^ azf,
                        axf ^ ayf ^ azc, axf ^ ayf ^ azf,
                    )
                    for c in range(8):
                        h0 = ((hs[c] & msk) + off).astype(jnp.int32)
                        idx_v[pl.ds(rb + c * 16, 16)] = h0
                return c2

            lax.fori_loop(0, GPC, grp_body, 0)

        def fire(idx_v, rows_v, sem):
            pltpu.async_copy(table_h.at[idx_v], rows_v, sem)

        def drain(idx_v, rows_v, sem):
            pltpu.make_async_copy(table_h.at[idx_v], rows_v, sem).wait()

        hi_msk = jnp.uint32(0xFFFF0000)

        def phase_b(ci, rows_v, frac_v):
            base = base_w + ci * CHUNK

            def grp_body(g, c2):
                for l in range(NLEV):
                    rb = (g * NLEV + l) * 8 * 16
                    f = []
                    for c in range(8):
                        w = lax.bitcast_convert_type(
                            rows_v[pl.ds(rb + c * 16, 16)], jnp.uint32)
                        f.append((
                            lax.bitcast_convert_type(w & hi_msk, jnp.float32),
                            lax.bitcast_convert_type(w << 16, jnp.float32)))
                    wb = g * 48 * 16
                    w0 = frac_v[pl.ds(wb + l * 16, 16)]
                    w1 = frac_v[pl.ds(wb + (16 + l) * 16, 16)]
                    w2 = frac_v[pl.ds(wb + (32 + l) * 16, 16)]

                    def lerp(a, b, w):
                        return b + (a - b) * w

                    for ft in range(FDIM):
                        f03 = lerp(f[0][ft], f[3][ft], w0)
                        f12 = lerp(f[1][ft], f[2][ft], w0)
                        f56 = lerp(f[5][ft], f[6][ft], w0)
                        f47 = lerp(f[4][ft], f[7][ft], w0)
                        f0312 = lerp(f03, f12, w1)
                        f4756 = lerp(f47, f56, w1)
                        e = lerp(f0312, f4756, w2)
                        enc_v[2 * l + ft, pl.ds(g * GRP, 16)] = e
                return c2

            lax.fori_loop(0, GPC, grp_body, 0)
            pltpu.sync_copy(enc_v, out_h.at[:, pl.ds(base, CHUNK)])

        phase_a(0, idx_a, frac_a)
        fire(idx_a, rows_a, sem_a)

        def pipe_body(i, carry):
            c1 = 2 * i + 1
            phase_a(c1, idx_b, frac_b)
            fire(idx_b, rows_b, sem_b)
            drain(idx_a, rows_a, sem_a)
            phase_b(c1 - 1, rows_a, frac_a)
            phase_a(c1 + 1, idx_a, frac_a)
            fire(idx_a, rows_a, sem_a)
            drain(idx_b, rows_b, sem_b)
            phase_b(c1, rows_b, frac_b)
            return carry

        lax.fori_loop(0, (NCHUNK - 2) // 2, pipe_body, 0)
        phase_a(NCHUNK - 1, idx_b, frac_b)
        fire(idx_b, rows_b, sem_b)
        drain(idx_a, rows_a, sem_a)
        phase_b(NCHUNK - 2, rows_a, frac_a)
        drain(idx_b, rows_b, sem_b)
        phase_b(NCHUNK - 1, rows_b, frac_b)

    return body(posx, posy, posz, scal_b, table_packed)


def _mlp_body(enc_ref, dir_ref, w1, b1, w2, b2, w3, b3, w4, b4, w5, b5,
              dens_ref, col_ref):
    enc_t = enc_ref[...]
    h = jnp.maximum(
        lax.dot_general(enc_t, w1[...], (((0,), (0,)), ((), ())),
                        preferred_element_type=jnp.float32) + b1[...], 0.0)
    dens = jnp.dot(h, w2[...], preferred_element_type=jnp.float32) + b2[...]
    dens_ref[...] = dens

    d = dir_ref[...]
    x = d[0:1, :]
    y = d[1:2, :]
    z = d[2:3, :]
    xx = x * x
    yy = y * y
    zz = z * z
    one = jnp.ones_like(x)
    sh_t = jnp.concatenate([
        0.28209479177387814 * one,
        0.4886025119029199 * y,
        0.4886025119029199 * z,
        0.4886025119029199 * x,
        1.0925484305920792 * x * y,
        1.0925484305920792 * y * z,
        0.9461746957575601 * zz - 0.31539156525252,
        1.0925484305920792 * x * z,
        0.5462742152960396 * (xx - yy),
        0.5900435899266435 * y * (3.0 * xx - yy),
        2.890611442640554 * x * y * z,
        0.4570457994644658 * y * (5.0 * zz - 1.0),
        0.3731763325901154 * z * (5.0 * zz - 3.0),
        0.4570457994644658 * x * (5.0 * zz - 1.0),
        1.445305721320277 * z * (xx - yy),
        0.5900435899266435 * x * (xx - 3.0 * yy),
    ], axis=0)

    w3m = w3[...]
    x1 = (jnp.dot(dens, w3m[0:16, :], preferred_element_type=jnp.float32)
          + lax.dot_general(sh_t, w3m[16:32, :], (((0,), (0,)), ((), ())),
                            preferred_element_type=jnp.float32)
          + b3[...])
    x1 = jnp.maximum(x1, 0.0)
    x2 = jnp.maximum(
        jnp.dot(x1, w4[...], preferred_element_type=jnp.float32) + b4[...], 0.0)
    x3 = jnp.dot(x2, w5[...], preferred_element_type=jnp.float32) + b5[...]
    col_ref[...] = jax.nn.sigmoid(x3)


def _mlp_tc(enc, dirt, W1, b1, W2, b2, W3, b3, W4, b4, W5p, b5p):
    BLK = 2048
    grid = (NPTS // BLK,)
    full = lambda shape: pl.BlockSpec(shape, lambda i: (0, 0))
    return pl.pallas_call(
        _mlp_body,
        grid=grid,
        in_specs=[
            pl.BlockSpec((ENC_DIM, BLK), lambda i: (0, i)),
            pl.BlockSpec((8, BLK), lambda i: (0, i)),
            full(W1.shape), full((1, 64)),
            full(W2.shape), full((1, 16)),
            full(W3.shape), full((1, 64)),
            full(W4.shape), full((1, 64)),
            full(W5p.shape), full((1, 8)),
        ],
        out_specs=[
            pl.BlockSpec((BLK, 16), lambda i: (i, 0)),
            pl.BlockSpec((BLK, 8), lambda i: (i, 0)),
        ],
        out_shape=[
            jax.ShapeDtypeStruct((NPTS, 16), jnp.float32),
            jax.ShapeDtypeStruct((NPTS, 8), jnp.float32),
        ],
    )(enc, dirt, W1, b1.reshape(1, 64), W2, b2.reshape(1, 16),
      W3, b3.reshape(1, 64), W4, b4.reshape(1, 64), W5p, b5p.reshape(1, 8))


def kernel(position, direction, hash_table, W1, b1, W2, b2, W3, b3, W4, b4,
           W5, b5):
    levels = jnp.arange(NLEV)
    growth = jnp.exp((jnp.log(MAX_RES) - jnp.log(MIN_RES)) / (NLEV - 1))
    scalings = jnp.floor(MIN_RES * growth ** levels).astype(jnp.float32)
    scal_b = jnp.repeat(scalings, 16)
    u0 = lax.bitcast_convert_type(
        hash_table[0].astype(jnp.bfloat16), jnp.uint16).astype(jnp.uint32)
    u1 = lax.bitcast_convert_type(
        hash_table[1].astype(jnp.bfloat16), jnp.uint16).astype(jnp.uint32)
    table_packed = lax.bitcast_convert_type((u0 << 16) | u1, jnp.int32)
    posx = position[:, 0]
    posy = position[:, 1]
    posz = position[:, 2]

    enc_t = _enc_sc(posx, posy, posz, scal_b, table_packed)

    dirt = jnp.pad(direction.T, ((0, 5), (0, 0)))
    W5p = jnp.pad(W5, ((0, 0), (0, 5)))
    b5p = jnp.pad(b5, (0, 5))
    dens, colp = _mlp_tc(enc_t, dirt, W1, b1, W2, b2, W3, b3, W4, b4, W5p, b5p)
    return dens, colp[:, :3]


# speedup vs baseline: 1.6979x; 1.0325x over previous
import functools

import jax
import jax.numpy as jnp
from jax import lax
from jax.experimental import pallas as pl
from jax.experimental.pallas import tpu as pltpu
from jax.experimental.pallas import tpu_sc as plsc

NPTS = 131072
NLEV = 16
TSIZE = 524288
FDIM = 2
ENC_DIM = NLEV * FDIM
MIN_RES = 16.0
MAX_RES = 1024.0

NW = 32
PTS_PER_W = NPTS // NW
GRP = 16
CHUNK = 128
GPC = CHUNK // GRP
NCHUNK = PTS_PER_W // CHUNK
ROWS = CHUNK * NLEV * 8

P1 = 2654435761
P2 = 805459861
MASK = TSIZE - 1


def _enc_sc(posx, posy, posz, scal_b, table_packed, npts):
    pts_per_w = npts // NW
    nchunk = pts_per_w // CHUNK
    mesh = plsc.VectorSubcoreMesh(core_axis_name="c", subcore_axis_name="s")

    @functools.partial(
        pl.kernel,
        out_type=jax.ShapeDtypeStruct((ENC_DIM, npts), jnp.float32),
        mesh=mesh,
        scratch_types=[
            pltpu.VMEM((CHUNK,), jnp.float32),
            pltpu.VMEM((CHUNK,), jnp.float32),
            pltpu.VMEM((CHUNK,), jnp.float32),
            pltpu.VMEM((NLEV * 16,), jnp.float32),
            pltpu.VMEM((ROWS,), jnp.int32),
            pltpu.VMEM((ROWS,), jnp.int32),
            pltpu.VMEM((ROWS,), jnp.int32),
            pltpu.VMEM((ROWS,), jnp.int32),
            pltpu.VMEM((CHUNK * 48,), jnp.float32),
            pltpu.VMEM((CHUNK * 48,), jnp.float32),
            pltpu.VMEM((ENC_DIM, CHUNK), jnp.float32),
            pltpu.SemaphoreType.DMA,
            pltpu.SemaphoreType.DMA,
        ],
    )
    def body(posx_h, posy_h, posz_h, scal_h, table_h, out_h,
             px, py, pz, scal_v, idx_a, idx_b, rows_a, rows_b,
             frac_a, frac_b, enc_v, sem_a, sem_b):
        wid = lax.axis_index("s") * 2 + lax.axis_index("c")
        base_w = wid * pts_per_w
        pltpu.sync_copy(scal_h, scal_v)
        iota = lax.iota(jnp.int32, 16)
        zcol = iota * 0
        ocol = zcol + 1

        def phase_a(ci, idx_v, frac_v):
            base = base_w + ci * CHUNK
            pltpu.sync_copy(posx_h.at[pl.ds(base, CHUNK)], px)
            pltpu.sync_copy(posy_h.at[pl.ds(base, CHUNK)], py)
            pltpu.sync_copy(posz_h.at[pl.ds(base, CHUNK)], pz)

            def grp_body(g, c2):
                xs = px[pl.ds(g * GRP, 16)]
                ys = py[pl.ds(g * GRP, 16)]
                zs = pz[pl.ds(g * GRP, 16)]
                for l in range(NLEV):
                    scal = scal_v[pl.ds(l * 16, 16)]
                    sx = xs * scal
                    sy = ys * scal
                    sz = zs * scal
                    fxi = sx.astype(jnp.int32)
                    fyi = sy.astype(jnp.int32)
                    fzi = sz.astype(jnp.int32)
                    fxf = fxi.astype(jnp.float32)
                    fyf = fyi.astype(jnp.float32)
                    fzf = fzi.astype(jnp.float32)
                    cxi = fxi + jnp.where(sx > fxf, ocol, zcol)
                    cyi = fyi + jnp.where(sy > fyf, ocol, zcol)
                    czi = fzi + jnp.where(sz > fzf, ocol, zcol)
                    fb = (g * 48 + l * 3) * 16
                    frac_v[pl.ds(fb, 16)] = sx - fxf
                    frac_v[pl.ds(fb + 16, 16)] = sy - fyf
                    frac_v[pl.ds(fb + 32, 16)] = sz - fzf
                    axc = cxi.astype(jnp.uint32)
                    axf = fxi.astype(jnp.uint32)
                    ayc = cyi.astype(jnp.uint32) * jnp.uint32(P1)
                    ayf = fyi.astype(jnp.uint32) * jnp.uint32(P1)
                    azc = czi.astype(jnp.uint32) * jnp.uint32(P2)
                    azf = fzi.astype(jnp.uint32) * jnp.uint32(P2)
                    off = jnp.uint32(l * TSIZE)
                    msk = jnp.uint32(MASK)
                    rb = (g * NLEV + l) * 8 * 16
                    hs = (
                        axc ^ ayc ^ azc, axc ^ ayc ^ azf,
                        axc ^ ayf ^ azc, axf ^ ayc ^ azc,
                        axc ^ ayf ^ azf, axf ^ ayc ^ azf,
                        axf ^ ayf ^ azc, axf ^ ayf ^ azf,
                    )
                    for c in range(8):
                        h0 = ((hs[c] & msk) + off).astype(jnp.int32)
                        idx_v[pl.ds(rb + c * 16, 16)] = h0
                return c2

            lax.fori_loop(0, GPC, grp_body, 0)

        def fire(idx_v, rows_v, sem):
            pltpu.async_copy(table_h.at[idx_v], rows_v, sem)

        def drain(idx_v, rows_v, sem):
            pltpu.make_async_copy(table_h.at[idx_v], rows_v, sem).wait()

        hi_msk = jnp.uint32(0xFFFF0000)

        def phase_b(ci, rows_v, frac_v):
            base = base_w + ci * CHUNK

            def grp_body(g, c2):
                for l in range(NLEV):
                    rb = (g * NLEV + l) * 8 * 16
                    f = []
                    for c in range(8):
                        w = lax.bitcast_convert_type(
                            rows_v[pl.ds(rb + c * 16, 16)], jnp.uint32)
                        f.append((
                            lax.bitcast_convert_type(w & hi_msk, jnp.float32),
                            lax.bitcast_convert_type(w << 16, jnp.float32)))
                    wb = g * 48 * 16
                    w0 = frac_v[pl.ds(wb + l * 16, 16)]
                    w1 = frac_v[pl.ds(wb + (16 + l) * 16, 16)]
                    w2 = frac_v[pl.ds(wb + (32 + l) * 16, 16)]

                    def lerp(a, b, w):
                        return b + (a - b) * w

                    for ft in range(FDIM):
                        f03 = lerp(f[0][ft], f[3][ft], w0)
                        f12 = lerp(f[1][ft], f[2][ft], w0)
                        f56 = lerp(f[5][ft], f[6][ft], w0)
                        f47 = lerp(f[4][ft], f[7][ft], w0)
                        f0312 = lerp(f03, f12, w1)
                        f4756 = lerp(f47, f56, w1)
                        e = lerp(f0312, f4756, w2)
                        enc_v[2 * l + ft, pl.ds(g * GRP, 16)] = e
                return c2

            lax.fori_loop(0, GPC, grp_body, 0)
            pltpu.sync_copy(enc_v, out_h.at[:, pl.ds(base, CHUNK)])

        phase_a(0, idx_a, frac_a)
        fire(idx_a, rows_a, sem_a)

        def pipe_body(i, carry):
            c1 = 2 * i + 1
            phase_a(c1, idx_b, frac_b)
            fire(idx_b, rows_b, sem_b)
            drain(idx_a, rows_a, sem_a)
            phase_b(c1 - 1, rows_a, frac_a)
            phase_a(c1 + 1, idx_a, frac_a)
            fire(idx_a, rows_a, sem_a)
            drain(idx_b, rows_b, sem_b)
            phase_b(c1, rows_b, frac_b)
            return carry

        lax.fori_loop(0, (nchunk - 2) // 2, pipe_body, 0)
        phase_a(nchunk - 1, idx_b, frac_b)
        fire(idx_b, rows_b, sem_b)
        drain(idx_a, rows_a, sem_a)
        phase_b(nchunk - 2, rows_a, frac_a)
        drain(idx_b, rows_b, sem_b)
        phase_b(nchunk - 1, rows_b, frac_b)

    return body(posx, posy, posz, scal_b, table_packed)


def _mlp_body(enc_ref, dir_ref, w1, b1, w2, b2, w3, b3, w4, b4, w5, b5,
              dens_ref, col_ref):
    enc_t = enc_ref[...]
    h = jnp.maximum(
        lax.dot_general(enc_t, w1[...], (((0,), (0,)), ((), ())),
                        preferred_element_type=jnp.float32) + b1[...], 0.0)
    dens = jnp.dot(h, w2[...], preferred_element_type=jnp.float32) + b2[...]
    dens_ref[...] = dens

    d = dir_ref[...]
    x = d[0:1, :]
    y = d[1:2, :]
    z = d[2:3, :]
    xx = x * x
    yy = y * y
    zz = z * z
    one = jnp.ones_like(x)
    sh_t = jnp.concatenate([
        0.28209479177387814 * one,
        0.4886025119029199 * y,
        0.4886025119029199 * z,
        0.4886025119029199 * x,
        1.0925484305920792 * x * y,
        1.0925484305920792 * y * z,
        0.9461746957575601 * zz - 0.31539156525252,
        1.0925484305920792 * x * z,
        0.5462742152960396 * (xx - yy),
        0.5900435899266435 * y * (3.0 * xx - yy),
        2.890611442640554 * x * y * z,
        0.4570457994644658 * y * (5.0 * zz - 1.0),
        0.3731763325901154 * z * (5.0 * zz - 3.0),
        0.4570457994644658 * x * (5.0 * zz - 1.0),
        1.445305721320277 * z * (xx - yy),
        0.5900435899266435 * x * (xx - 3.0 * yy),
    ], axis=0)

    w3m = w3[...]
    x1 = (jnp.dot(dens, w3m[0:16, :], preferred_element_type=jnp.float32)
          + lax.dot_general(sh_t, w3m[16:32, :], (((0,), (0,)), ((), ())),
                            preferred_element_type=jnp.float32)
          + b3[...])
    x1 = jnp.maximum(x1, 0.0)
    x2 = jnp.maximum(
        jnp.dot(x1, w4[...], preferred_element_type=jnp.float32) + b4[...], 0.0)
    x3 = jnp.dot(x2, w5[...], preferred_element_type=jnp.float32) + b5[...]
    col_ref[...] = jax.nn.sigmoid(x3)


def _mlp_tc(enc, dirt, W1, b1, W2, b2, W3, b3, W4, b4, W5p, b5p, npts):
    BLK = 2048
    grid = (npts // BLK,)
    full = lambda shape: pl.BlockSpec(shape, lambda i: (0, 0))
    return pl.pallas_call(
        _mlp_body,
        grid=grid,
        in_specs=[
            pl.BlockSpec((ENC_DIM, BLK), lambda i: (0, i)),
            pl.BlockSpec((8, BLK), lambda i: (0, i)),
            full(W1.shape), full((1, 64)),
            full(W2.shape), full((1, 16)),
            full(W3.shape), full((1, 64)),
            full(W4.shape), full((1, 64)),
            full(W5p.shape), full((1, 8)),
        ],
        out_specs=[
            pl.BlockSpec((BLK, 16), lambda i: (i, 0)),
            pl.BlockSpec((BLK, 8), lambda i: (i, 0)),
        ],
        out_shape=[
            jax.ShapeDtypeStruct((npts, 16), jnp.float32),
            jax.ShapeDtypeStruct((npts, 8), jnp.float32),
        ],
    )(enc, dirt, W1, b1.reshape(1, 64), W2, b2.reshape(1, 16),
      W3, b3.reshape(1, 64), W4, b4.reshape(1, 64), W5p, b5p.reshape(1, 8))


def kernel(position, direction, hash_table, W1, b1, W2, b2, W3, b3, W4, b4,
           W5, b5):
    levels = jnp.arange(NLEV)
    growth = jnp.exp((jnp.log(MAX_RES) - jnp.log(MIN_RES)) / (NLEV - 1))
    scalings = jnp.floor(MIN_RES * growth ** levels).astype(jnp.float32)
    scal_b = jnp.repeat(scalings, 16)
    u0 = lax.bitcast_convert_type(
        hash_table[0].astype(jnp.bfloat16), jnp.uint16).astype(jnp.uint32)
    u1 = lax.bitcast_convert_type(
        hash_table[1].astype(jnp.bfloat16), jnp.uint16).astype(jnp.uint32)
    table_packed = lax.bitcast_convert_type((u0 << 16) | u1, jnp.int32)
    posx = position[:, 0]
    posy = position[:, 1]
    posz = position[:, 2]

    dirt = jnp.pad(direction.T, ((0, 5), (0, 0)))
    W5p = jnp.pad(W5, ((0, 0), (0, 5)))
    b5p = jnp.pad(b5, (0, 5))

    H = NPTS // 2
    dens, colp = [], []
    for i in range(2):
        s = slice(i * H, (i + 1) * H)
        enc_t = _enc_sc(posx[s], posy[s], posz[s], scal_b, table_packed, H)
        d, c = _mlp_tc(enc_t, dirt[:, s], W1, b1, W2, b2, W3, b3, W4, b4,
                       W5p, b5p, H)
        dens.append(d)
        colp.append(c)
    dens = jnp.concatenate(dens, axis=0)
    colp = jnp.concatenate(colp, axis=0)
    return dens, colp[:, :3]
